# baseline (speedup 1.0000x reference)
import functools

import jax
import jax.numpy as jnp
from jax import lax
from jax.experimental import pallas as pl
from jax.experimental.pallas import tpu as pltpu
from jax.experimental.pallas import tpu_sc as plsc

_VOCAB = 1000001
_DIM = 96
_BATCH = 16384

_INFO = plsc.get_sparse_core_info()
_NC = _INFO.num_cores
_NS = _INFO.num_subcores
_NW = _NC * _NS
_B_PER_W = _BATCH // _NW
_HALF = _B_PER_W // 2


@functools.partial(
    pl.kernel,
    mesh=plsc.VectorSubcoreMesh(core_axis_name="c", subcore_axis_name="s"),
    out_type=jax.ShapeDtypeStruct((_BATCH, _DIM), jnp.float32),
    scratch_types=[
        pltpu.VMEM((_B_PER_W,), jnp.int32),
        pltpu.VMEM((_HALF, _DIM), jnp.float32),
        pltpu.SemaphoreType.DMA,
        pltpu.SemaphoreType.DMA,
    ],
)
def _sc_gather(idx_hbm, table_hbm, out_hbm, idx_v, rows_v, sem_h, sem_v):
    wid = lax.axis_index("s") * _NC + lax.axis_index("c")
    base = wid * _B_PER_W
    pltpu.sync_copy(idx_hbm.at[pl.ds(base, _B_PER_W)], idx_v)

    def body(blk):
        vec = idx_v[pl.ds(blk * 16, 16)]
        vec2 = idx_v[pl.ds(_HALF + blk * 16, 16)]
        for l in range(16):
            j = blk * 16 + l
            pltpu.make_async_copy(
                table_hbm.at[vec[l]], out_hbm.at[base + j], sem_h
            ).start()
            pltpu.make_async_copy(
                table_hbm.at[vec2[l]], rows_v.at[j], sem_v
            ).start()

    pl.loop(0, _HALF // 16)(body)
    pltpu.make_async_copy(
        out_hbm.at[pl.ds(base, _HALF)], out_hbm.at[pl.ds(base, _HALF)], sem_h
    ).wait()
    pltpu.make_async_copy(out_hbm.at[pl.ds(0, _HALF)], rows_v, sem_v).wait()
    pltpu.sync_copy(rows_v, out_hbm.at[pl.ds(base + _HALF, _HALF)])


def kernel(broadcaster, table):
    return _sc_gather(broadcaster, table)


# speedup vs baseline: 1.2010x; 1.2010x over previous
import functools

import jax
import jax.numpy as jnp
from jax import lax
from jax.experimental import pallas as pl
from jax.experimental.pallas import tpu as pltpu
from jax.experimental.pallas import tpu_sc as plsc

_VOCAB = 1000001
_DIM = 96
_BATCH = 16384

_INFO = plsc.get_sparse_core_info()
_NC = _INFO.num_cores
_NS = _INFO.num_subcores
_NW = _NC * _NS
_B_PER_W = _BATCH // _NW
_B_PER_C = _NS * _B_PER_W


@functools.partial(
    pl.kernel,
    mesh=plsc.VectorSubcoreMesh(core_axis_name="c", subcore_axis_name="s"),
    out_type=jax.ShapeDtypeStruct((_BATCH, _DIM), jnp.float32),
    scratch_types=[
        pltpu.VMEM((_B_PER_W,), jnp.int32),
        pltpu.VMEM_SHARED((_B_PER_C, _DIM), jnp.float32),
        pltpu.SemaphoreType.DMA,
    ],
)
def _sc_gather(idx_hbm, table_hbm, out_hbm, idx_v, shared_v, sem):
    sid = lax.axis_index("s")
    wid = sid * _NC + lax.axis_index("c")
    base = wid * _B_PER_W
    sbase = sid * _B_PER_W
    pltpu.sync_copy(idx_hbm.at[pl.ds(base, _B_PER_W)], idx_v)

    def body(blk):
        vec = idx_v[pl.ds(blk * 16, 16)]
        for l in range(16):
            i = vec[l]
            pltpu.make_async_copy(
                table_hbm.at[i], shared_v.at[sbase + blk * 16 + l], sem
            ).start()

    pl.loop(0, _B_PER_W // 16)(body)
    pltpu.make_async_copy(
        out_hbm.at[pl.ds(0, _B_PER_W)],
        shared_v.at[pl.ds(sbase, _B_PER_W)],
        sem,
    ).wait()
    pltpu.sync_copy(
        shared_v.at[pl.ds(sbase, _B_PER_W)], out_hbm.at[pl.ds(base, _B_PER_W)]
    )


def kernel(broadcaster, table):
    return _sc_gather(broadcaster, table)


# speedup vs baseline: 1.2917x; 1.0755x over previous
import functools

import jax
import jax.numpy as jnp
from jax import lax
from jax.experimental import pallas as pl
from jax.experimental.pallas import tpu as pltpu
from jax.experimental.pallas import tpu_sc as plsc

_VOCAB = 1000001
_DIM = 96
_BATCH = 16384

_INFO = plsc.get_sparse_core_info()
_NC = _INFO.num_cores
_NS = _INFO.num_subcores
_NW = _NC * _NS
_B_PER_W = _BATCH // _NW


@functools.partial(
    pl.kernel,
    mesh=plsc.VectorSubcoreMesh(core_axis_name="c", subcore_axis_name="s"),
    out_type=jax.ShapeDtypeStruct((_BATCH, _DIM), jnp.float32),
    scratch_types=[
        pltpu.VMEM((_B_PER_W,), jnp.int32),
        pltpu.VMEM((_B_PER_W, _DIM), jnp.float32),
        pltpu.SemaphoreType.DMA,
    ],
)
def _sc_gather(idx_hbm, table_hbm, out_hbm, idx_v, rows_v, sem):
    wid = lax.axis_index("s") * _NC + lax.axis_index("c")
    base = wid * _B_PER_W
    pltpu.sync_copy(idx_hbm.at[pl.ds(base, _B_PER_W)], idx_v)

    def body(blk):
        vec = idx_v[pl.ds(blk * 16, 16)]
        for l in range(16):
            i = vec[l]
            pltpu.make_async_copy(
                table_hbm.at[i], rows_v.at[blk * 16 + l], sem
            ).start()

    pl.loop(0, _B_PER_W // 16)(body)
    pltpu.make_async_copy(out_hbm.at[pl.ds(0, _B_PER_W)], rows_v, sem).wait()
    pltpu.sync_copy(rows_v, out_hbm.at[pl.ds(base, _B_PER_W)])


def kernel(broadcaster, table):
    return _sc_gather(broadcaster, table)
